# baseline (speedup 1.0000x reference)
import functools

import jax
import jax.numpy as jnp
from jax import lax
from jax.experimental import pallas as pl
from jax.experimental.pallas import tpu as pltpu
from jax.experimental.pallas import tpu_sc as plsc

BATCH = 16384
TABLE = 1001
NUM_CORES = 2
NUM_SUBCORES = 16
NUM_WORKERS = NUM_CORES * NUM_SUBCORES
ELEMS = BATCH // NUM_WORKERS
LANES = 16
MAGIC = 12582912.0


def _sc_body(t_hbm, tab_hbm, out_hbm, t_v, tab_v, out_v, sem_tab, sem_t):
    wid = lax.axis_index("s") * NUM_CORES + lax.axis_index("c")
    base = wid * ELEMS
    cp_tab = pltpu.async_copy(tab_hbm, tab_v, sem_tab)
    cp_t = pltpu.async_copy(t_hbm.at[pl.ds(base, ELEMS)], t_v, sem_t)
    cp_tab.wait()
    cp_t.wait()
    @plsc.parallel_loop(0, ELEMS // LANES, unroll=4)
    def _(i):
        tv = t_v[pl.ds(i * LANES, LANES)]
        y = (tv * 1000.0 + MAGIC) - MAGIC
        idx = y.astype(jnp.int32)
        out_v[pl.ds(i * LANES, LANES)] = plsc.load_gather(tab_v, [idx])
    pltpu.sync_copy(out_v, out_hbm.at[pl.ds(base, ELEMS)])


@jax.jit
def kernel(t, gamma_table):
    mesh = plsc.VectorSubcoreMesh(
        core_axis_name="c", subcore_axis_name="s", num_cores=NUM_CORES
    )
    run = pl.kernel(
        _sc_body,
        mesh=mesh,
        out_type=jax.ShapeDtypeStruct((BATCH,), jnp.float32),
        scratch_types=[
            pltpu.VMEM((ELEMS,), jnp.float32),
            pltpu.VMEM((TABLE,), jnp.float32),
            pltpu.VMEM((ELEMS,), jnp.float32),
            pltpu.SemaphoreType.DMA,
            pltpu.SemaphoreType.DMA,
        ],
        compiler_params=pltpu.CompilerParams(needs_layout_passes=False),
    )
    return run(t, gamma_table)


# speedup vs baseline: 1.0910x; 1.0910x over previous
import functools

import jax
import jax.numpy as jnp
from jax import lax
from jax.experimental import pallas as pl
from jax.experimental.pallas import tpu as pltpu
from jax.experimental.pallas import tpu_sc as plsc

BATCH = 16384
TABLE = 1001
NUM_CORES = 1
NUM_SUBCORES = 16
NUM_WORKERS = NUM_CORES * NUM_SUBCORES
ELEMS = BATCH // NUM_WORKERS
LANES = 16
MAGIC = 12582912.0


def _sc_body(t_hbm, tab_hbm, out_hbm, t_v, tab_v, out_v, sem_tab, sem_t):
    wid = lax.axis_index("s") * NUM_CORES + lax.axis_index("c")
    base = wid * ELEMS
    cp_tab = pltpu.async_copy(tab_hbm, tab_v, sem_tab)
    cp_t = pltpu.async_copy(t_hbm.at[pl.ds(base, ELEMS)], t_v, sem_t)
    cp_tab.wait()
    cp_t.wait()
    @plsc.parallel_loop(0, ELEMS // LANES, unroll=4)
    def _(i):
        tv = t_v[pl.ds(i * LANES, LANES)]
        y = (tv * 1000.0 + MAGIC) - MAGIC
        idx = y.astype(jnp.int32)
        out_v[pl.ds(i * LANES, LANES)] = plsc.load_gather(tab_v, [idx])
    pltpu.sync_copy(out_v, out_hbm.at[pl.ds(base, ELEMS)])


@jax.jit
def kernel(t, gamma_table):
    mesh = plsc.VectorSubcoreMesh(
        core_axis_name="c", subcore_axis_name="s", num_cores=NUM_CORES
    )
    run = pl.kernel(
        _sc_body,
        mesh=mesh,
        out_type=jax.ShapeDtypeStruct((BATCH,), jnp.float32),
        scratch_types=[
            pltpu.VMEM((ELEMS,), jnp.float32),
            pltpu.VMEM((TABLE,), jnp.float32),
            pltpu.VMEM((ELEMS,), jnp.float32),
            pltpu.SemaphoreType.DMA,
            pltpu.SemaphoreType.DMA,
        ],
        compiler_params=pltpu.CompilerParams(needs_layout_passes=False),
    )
    return run(t, gamma_table)
